# Initial kernel scaffold; baseline (speedup 1.0000x reference)
import functools

import jax
import jax.numpy as jnp
from jax import lax
from jax.experimental import pallas as pl
from jax.experimental.pallas import tpu as pltpu
from jax.experimental.pallas import tpu_sc as plsc

N_NODES = 10000
D = 128
NC = 2
NS = 16
LANES = 16
NW = NC * NS
ACCW = 144
NPAD = 10112
STRIPE = NPAD // NS


def _mm_body(x_ref, wt_ref, am_ref, z_ref, alpha_ref):
    z = jnp.dot(x_ref[...], wt_ref[...], preferred_element_type=jnp.float32)
    z_ref[...] = z
    alpha_ref[...] = jnp.dot(z, am_ref[...], preferred_element_type=jnp.float32)


def _sc_body(z_hbm, alpha_hbm, src_hbm, dst_hbm, out_hbm,
             alpha_v, src_v, dst_v, rows_v, w_v, acc_s):
    cid = lax.axis_index("c")
    sid = lax.axis_index("s")
    wid = cid * NS + sid
    nchunks, chunk = src_v.shape

    zeros16 = jnp.zeros((LANES,), jnp.float32)

    def zero_rows(c, _):
        for k in range(ACCW // LANES):
            rows_v[c, pl.ds(k * LANES, LANES)] = zeros16
        return 0
    lax.fori_loop(0, chunk, zero_rows, 0)
    base = sid * STRIPE
    for t in range(8):
        cnt = chunk if t < 7 else STRIPE - 7 * chunk
        pltpu.sync_copy(rows_v.at[pl.ds(0, cnt)],
                        acc_s.at[pl.ds(base + t * chunk, cnt)])

    pltpu.sync_copy(src_hbm.at[wid], src_v)
    pltpu.sync_copy(dst_hbm.at[wid], dst_v)
    pltpu.sync_copy(alpha_hbm, alpha_v)

    plsc.subcore_barrier()

    lane_iota = lax.iota(jnp.int32, LANES)
    zeros_i = jnp.zeros((LANES,), jnp.int32)
    ones_i = jnp.ones((LANES,), jnp.int32)
    lane0 = lane_iota == 0

    def chunk_body(j, _):
        pltpu.sync_copy(z_hbm.at[src_v.at[j]], rows_v.at[:, pl.ds(0, D)])

        for i in range(chunk // LANES):
            sl = pl.ds(i * LANES, LANES)
            sidx = src_v[j, sl]
            didx = dst_v[j, sl]
            a1 = plsc.load_gather(alpha_v, [sidx, zeros_i])
            a2 = plsc.load_gather(alpha_v, [didx, ones_i])
            e = a1 + a2
            e = jnp.where(e >= 0.0, e, 0.01 * e)
            w_v[sl] = jnp.exp(e)

        def scale_row(c, _):
            wc = plsc.load_gather(w_v, [jnp.full((LANES,), c, jnp.int32)])
            for k in range(D // LANES):
                sl = pl.ds(k * LANES, LANES)
                rows_v[c, sl] = rows_v[c, sl] * wc
            rows_v[c, pl.ds(D, LANES)] = jnp.where(lane0, wc, 0.0)
            return 0
        lax.fori_loop(0, chunk, scale_row, 0)

        pltpu.sync_copy(rows_v, acc_s.at[dst_v.at[j]], add=True)
        return 0

    lax.fori_loop(0, nchunks, chunk_body, 0)

    plsc.subcore_barrier()

    pltpu.sync_copy(acc_s.at[pl.ds(base, STRIPE)],
                    out_hbm.at[cid, pl.ds(base, STRIPE)])


def _combine_body(a0_ref, a1_ref, o_ref):
    s = a0_ref[0] + a1_ref[0]
    num = s[:, :D]
    den = s[:, D:D + 1]
    o_ref[...] = num / den


def kernel(x, edge_index, W, A):
    n, d_in = x.shape
    d_out = W.shape[0]
    e_total = edge_index.shape[1]
    epw = e_total // NW
    chunk = 80
    nchunks = epw // chunk

    wt = W.T
    a_mat = jnp.stack([A[0, :d_out], A[0, d_out:]], axis=1)

    z, alpha = pl.pallas_call(
        _mm_body,
        out_shape=[
            jax.ShapeDtypeStruct((n, d_out), jnp.float32),
            jax.ShapeDtypeStruct((n, 2), jnp.float32),
        ],
    )(x, wt, a_mat)

    src = edge_index[0].astype(jnp.int32).reshape(NW, nchunks, chunk)
    dst = edge_index[1].astype(jnp.int32).reshape(NW, nchunks, chunk)

    sc = pl.kernel(
        _sc_body,
        out_type=jax.ShapeDtypeStruct((NC, NPAD, ACCW), jnp.float32),
        mesh=plsc.VectorSubcoreMesh(
            core_axis_name="c", subcore_axis_name="s",
            num_cores=NC, num_subcores=NS),
        scratch_types=[
            pltpu.VMEM((n, 2), jnp.float32),
            pltpu.VMEM((nchunks, chunk), jnp.int32),
            pltpu.VMEM((nchunks, chunk), jnp.int32),
            pltpu.VMEM((chunk, ACCW), jnp.float32),
            pltpu.VMEM((chunk,), jnp.float32),
            pltpu.VMEM_SHARED((NPAD, ACCW), jnp.float32),
        ],
    )
    acc = sc(z, alpha, src, dst)

    blk = 1000
    h = pl.pallas_call(
        _combine_body,
        grid=(n // blk,),
        in_specs=[
            pl.BlockSpec((1, blk, ACCW), lambda i: (0, i, 0)),
            pl.BlockSpec((1, blk, ACCW), lambda i: (1, i, 0)),
        ],
        out_specs=pl.BlockSpec((blk, d_out), lambda i: (i, 0)),
        out_shape=jax.ShapeDtypeStruct((n, d_out), jnp.float32),
    )(acc, acc)
    return h


# speedup vs baseline: 8.6060x; 8.6060x over previous
import jax
import jax.numpy as jnp
from jax import lax
from jax.experimental import pallas as pl
from jax.experimental.pallas import tpu as pltpu
from jax.experimental.pallas import tpu_sc as plsc

D = 128
NC = 2
NS = 16
LANES = 16
NW = NC * NS
NPAD = 10112
STRIPE = NPAD // NS
DENP = 10240


def _mm_body(x_ref, wt_ref, a1_ref, a2_ref, z_ref, al1_ref, al2_ref):
    z = jnp.dot(x_ref[...], wt_ref[...], preferred_element_type=jnp.float32)
    z_ref[...] = z
    al1_ref[...] = jnp.dot(z, a1_ref[...], preferred_element_type=jnp.float32)
    al2_ref[...] = jnp.dot(z, a2_ref[...], preferred_element_type=jnp.float32)


def _sc_body(z_hbm, al1_hbm, al2_hbm, src_hbm, dst_hbm, acc_hbm, den_hbm,
             al1_v, al2_v, sidx_v, didx_v, rows_v, w_v, den_v, acc_s):
    cid = lax.axis_index("c")
    sid = lax.axis_index("s")
    wid = cid * NS + sid
    nchunks = src_hbm.shape[1]
    chunk = src_hbm.shape[2]

    zeros16 = jnp.zeros((LANES,), jnp.float32)

    def zero_rows(c, _):
        for k in range(D // LANES):
            rows_v[c, pl.ds(k * LANES, LANES)] = zeros16
        return 0
    lax.fori_loop(0, chunk, zero_rows, 0)

    def zero_den(c, _):
        den_v[pl.ds(c * LANES, LANES)] = zeros16
        return 0
    lax.fori_loop(0, DENP // LANES, zero_den, 0)

    base = sid * STRIPE
    off = 0
    while off < STRIPE:
        cnt = min(chunk, STRIPE - off)
        pltpu.sync_copy(rows_v.at[pl.ds(0, cnt)],
                        acc_s.at[pl.ds(base + off, cnt)])
        off += cnt

    pltpu.sync_copy(al1_hbm, al1_v)
    pltpu.sync_copy(al2_hbm, al2_v)

    plsc.subcore_barrier()

    def chunk_body(j, _):
        pltpu.sync_copy(src_hbm.at[wid, j], sidx_v)
        pltpu.sync_copy(dst_hbm.at[wid, j], didx_v)
        pltpu.sync_copy(z_hbm.at[sidx_v], rows_v)

        for i in range(chunk // LANES):
            sl = pl.ds(i * LANES, LANES)
            sidx = sidx_v[sl]
            didx = didx_v[sl]
            a1 = plsc.load_gather(al1_v, [sidx])
            a2 = plsc.load_gather(al2_v, [didx])
            e = a1 + a2
            e = jnp.where(e >= 0.0, e, 0.01 * e)
            w = jnp.exp(e)
            w_v[sl] = w
            plsc.addupdate_scatter(den_v, [didx], w)

        def scale_row(c, _):
            wc = plsc.load_gather(w_v, [jnp.full((LANES,), c, jnp.int32)])
            for k in range(D // LANES):
                sl = pl.ds(k * LANES, LANES)
                rows_v[c, sl] = rows_v[c, sl] * wc
            return 0
        lax.fori_loop(0, chunk, scale_row, 0)

        pltpu.sync_copy(rows_v, acc_s.at[didx_v], add=True)
        return 0

    lax.fori_loop(0, nchunks, chunk_body, 0)

    pltpu.sync_copy(den_v, den_hbm.at[wid])

    plsc.subcore_barrier()

    pltpu.sync_copy(acc_s.at[pl.ds(base, STRIPE)],
                    acc_hbm.at[cid, pl.ds(base, STRIPE)])


def _combine_body(a0_ref, a1_ref, dp_ref, o_ref):
    s = a0_ref[0] + a1_ref[0]
    den = jnp.sum(dp_ref[...], axis=0)
    o_ref[...] = s / den


def kernel(x, edge_index, W, A):
    n, d_in = x.shape
    d_out = W.shape[0]
    e_total = edge_index.shape[1]
    epw = e_total // NW
    chunk = 80
    nchunks = epw // chunk

    wt = W.T
    a1 = A[0, :d_out]
    a2 = A[0, d_out:]

    z, al1, al2 = pl.pallas_call(
        _mm_body,
        out_shape=[
            jax.ShapeDtypeStruct((n, d_out), jnp.float32),
            jax.ShapeDtypeStruct((n,), jnp.float32),
            jax.ShapeDtypeStruct((n,), jnp.float32),
        ],
    )(x, wt, a1, a2)

    src = edge_index[0].astype(jnp.int32).reshape(NW, nchunks, chunk)
    dst = edge_index[1].astype(jnp.int32).reshape(NW, nchunks, chunk)

    sc = pl.kernel(
        _sc_body,
        out_type=[
            jax.ShapeDtypeStruct((NC, NPAD, D), jnp.float32),
            jax.ShapeDtypeStruct((NW, DENP), jnp.float32),
        ],
        mesh=plsc.VectorSubcoreMesh(
            core_axis_name="c", subcore_axis_name="s",
            num_cores=NC, num_subcores=NS),
        compiler_params=pltpu.CompilerParams(needs_layout_passes=False),
        scratch_types=[
            pltpu.VMEM((n,), jnp.float32),
            pltpu.VMEM((n,), jnp.float32),
            pltpu.VMEM((chunk,), jnp.int32),
            pltpu.VMEM((chunk,), jnp.int32),
            pltpu.VMEM((chunk, D), jnp.float32),
            pltpu.VMEM((chunk,), jnp.float32),
            pltpu.VMEM((DENP,), jnp.float32),
            pltpu.VMEM_SHARED((NPAD, D), jnp.float32),
        ],
    )
    acc, den_part = sc(z, al1, al2, src, dst)
    den3 = den_part.reshape(NW, DENP, 1)

    blk = 1000
    h = pl.pallas_call(
        _combine_body,
        grid=(n // blk,),
        in_specs=[
            pl.BlockSpec((1, blk, D), lambda i: (0, i, 0)),
            pl.BlockSpec((1, blk, D), lambda i: (1, i, 0)),
            pl.BlockSpec((NW, blk, 1), lambda i: (0, i, 0)),
        ],
        out_specs=pl.BlockSpec((blk, d_out), lambda i: (i, 0)),
        out_shape=jax.ShapeDtypeStruct((n, d_out), jnp.float32),
    )(acc, acc, den3)
    return h


# speedup vs baseline: 9.7921x; 1.1378x over previous
import jax
import jax.numpy as jnp
from jax import lax
from jax.experimental import pallas as pl
from jax.experimental.pallas import tpu as pltpu
from jax.experimental.pallas import tpu_sc as plsc

D = 128
NC = 2
NS = 16
LANES = 16
NW = NC * NS
NPAD = 10112
STRIPE = NPAD // NS
DENP = 10112
EPW = 10112
CHUNK = 64
NCHUNKS = EPW // CHUNK


def _mm_body(x_ref, wt_ref, a1_ref, a2_ref, z_ref, al1_ref, al2_ref):
    z = jnp.dot(x_ref[...], wt_ref[...], preferred_element_type=jnp.float32)
    z_ref[...] = z
    al1_ref[...] = jnp.dot(z, a1_ref[...], preferred_element_type=jnp.float32)
    al2_ref[...] = jnp.dot(z, a2_ref[...], preferred_element_type=jnp.float32)


def _sc_body(z_hbm, al1_hbm, al2_hbm, src_hbm, dst_hbm, acc_hbm, den_hbm,
             al1_v, al2_v, sidx_v, didx_v, rows_v, w_v, den_v,
             gsem0, gsem1, isem0, isem1, acc_s):
    cid = lax.axis_index("c")
    sid = lax.axis_index("s")
    wid = cid * NS + sid
    nchunks = src_hbm.shape[1]
    chunk = src_hbm.shape[2]
    gsems = (gsem0, gsem1)
    isems = (isem0, isem1)

    zeros16 = jnp.zeros((LANES,), jnp.float32)

    def zero_rows(c, _):
        for k in range(D // LANES):
            rows_v[0, c, pl.ds(k * LANES, LANES)] = zeros16
        return 0
    lax.fori_loop(0, chunk, zero_rows, 0)

    def zero_den(c, _):
        den_v[pl.ds(c * LANES, LANES)] = zeros16
        return 0
    lax.fori_loop(0, DENP // LANES, zero_den, 0)

    base = sid * STRIPE
    off = 0
    while off < STRIPE:
        cnt = min(chunk, STRIPE - off)
        pltpu.sync_copy(rows_v.at[0, pl.ds(0, cnt)],
                        acc_s.at[pl.ds(base + off, cnt)])
        off += cnt

    pltpu.sync_copy(al1_hbm, al1_v)
    pltpu.sync_copy(al2_hbm, al2_v)
    pltpu.sync_copy(src_hbm.at[wid, 0], sidx_v.at[0])
    pltpu.sync_copy(dst_hbm.at[wid, 0], didx_v.at[0])
    pltpu.async_copy(src_hbm.at[wid, 1], sidx_v.at[1], isem1)
    pltpu.async_copy(dst_hbm.at[wid, 1], didx_v.at[1], isem1)
    pltpu.async_copy(z_hbm.at[sidx_v.at[0]], rows_v.at[0], gsem0)

    plsc.subcore_barrier()

    def pipe_step(j, b, last):
        nb = 1 - b
        if not last:
            pltpu.make_async_copy(src_hbm.at[wid, j + 1],
                                  sidx_v.at[nb], isems[nb]).wait()
            pltpu.make_async_copy(dst_hbm.at[wid, j + 1],
                                  didx_v.at[nb], isems[nb]).wait()
            pltpu.async_copy(z_hbm.at[sidx_v.at[nb]], rows_v.at[nb],
                             gsems[nb])

        pltpu.make_async_copy(z_hbm.at[sidx_v.at[b]], rows_v.at[b],
                              gsems[b]).wait()

        for i in range(chunk // LANES):
            sl = pl.ds(i * LANES, LANES)
            sidx = sidx_v[b, sl]
            didx = didx_v[b, sl]
            a1 = plsc.load_gather(al1_v, [sidx])
            a2 = plsc.load_gather(al2_v, [didx])
            e = a1 + a2
            e = jnp.where(e >= 0.0, e, 0.01 * e)
            w = jnp.exp(e)
            w_v[sl] = w
            plsc.addupdate_scatter(den_v, [didx], w)

        def scale_row(c, _):
            wc = plsc.load_gather(w_v, [jnp.full((LANES,), c, jnp.int32)])
            for k in range(D // LANES):
                sl = pl.ds(k * LANES, LANES)
                rows_v[b, c, sl] = rows_v[b, c, sl] * wc
            return 0
        lax.fori_loop(0, chunk, scale_row, 0)

        pltpu.sync_copy(rows_v.at[b], acc_s.at[didx_v.at[b]], add=True)

        if not last:
            @pl.when(jnp.asarray(j) + 2 < nchunks)
            def _():
                pltpu.async_copy(src_hbm.at[wid, j + 2], sidx_v.at[b],
                                 isems[b])
                pltpu.async_copy(dst_hbm.at[wid, j + 2], didx_v.at[b],
                                 isems[b])

    def pair_body(j2, _):
        j = 2 * j2
        pipe_step(j, 0, False)
        pipe_step(j + 1, 1, False)
        return 0

    pairs = (nchunks - 1) // 2
    lax.fori_loop(0, pairs, pair_body, 0)
    for j in range(2 * pairs, nchunks):
        pipe_step(j, j % 2, j == nchunks - 1)

    pltpu.sync_copy(den_v, den_hbm.at[wid])

    plsc.subcore_barrier()

    pltpu.sync_copy(acc_s.at[pl.ds(base, STRIPE)],
                    acc_hbm.at[cid, pl.ds(base, STRIPE)])


def _combine_body(a0_ref, a1_ref, dp_ref, o_ref):
    s = a0_ref[0] + a1_ref[0]
    den = jnp.sum(dp_ref[...], axis=0)
    o_ref[...] = s / den


def kernel(x, edge_index, W, A):
    n, d_in = x.shape
    d_out = W.shape[0]
    e_total = edge_index.shape[1]
    nchunks = NCHUNKS
    chunk = CHUNK

    wt = W.T
    a1 = A[0, :d_out]
    a2 = A[0, d_out:]

    z, al1, al2 = pl.pallas_call(
        _mm_body,
        out_shape=[
            jax.ShapeDtypeStruct((n, d_out), jnp.float32),
            jax.ShapeDtypeStruct((n,), jnp.float32),
            jax.ShapeDtypeStruct((n,), jnp.float32),
        ],
    )(x, wt, a1, a2)

    npad_e = NW * EPW - e_total
    al1p = jnp.concatenate([al1, jnp.zeros((DENP - n,), jnp.float32)])
    al2p = jnp.concatenate([al2, jnp.zeros((DENP - n,), jnp.float32)])
    src = jnp.concatenate(
        [edge_index[0].astype(jnp.int32),
         jnp.zeros((npad_e,), jnp.int32)]).reshape(NW, nchunks, chunk)
    dst = jnp.concatenate(
        [edge_index[1].astype(jnp.int32),
         jnp.full((npad_e,), DENP - 1, jnp.int32)]).reshape(NW, nchunks, chunk)

    sc = pl.kernel(
        _sc_body,
        out_type=[
            jax.ShapeDtypeStruct((NC, NPAD, D), jnp.float32),
            jax.ShapeDtypeStruct((NW, DENP), jnp.float32),
        ],
        mesh=plsc.VectorSubcoreMesh(
            core_axis_name="c", subcore_axis_name="s",
            num_cores=NC, num_subcores=NS),
        compiler_params=pltpu.CompilerParams(needs_layout_passes=False),
        scratch_types=[
            pltpu.VMEM((DENP,), jnp.float32),
            pltpu.VMEM((DENP,), jnp.float32),
            pltpu.VMEM((2, chunk), jnp.int32),
            pltpu.VMEM((2, chunk), jnp.int32),
            pltpu.VMEM((2, chunk, D), jnp.float32),
            pltpu.VMEM((chunk,), jnp.float32),
            pltpu.VMEM((DENP,), jnp.float32),
            pltpu.SemaphoreType.DMA,
            pltpu.SemaphoreType.DMA,
            pltpu.SemaphoreType.DMA,
            pltpu.SemaphoreType.DMA,
            pltpu.VMEM_SHARED((NPAD, D), jnp.float32),
        ],
    )
    acc, den_part = sc(z, al1p, al2p, src, dst)
    den3 = den_part.reshape(NW, DENP, 1)

    blk = 1000
    h = pl.pallas_call(
        _combine_body,
        grid=(n // blk,),
        in_specs=[
            pl.BlockSpec((1, blk, D), lambda i: (0, i, 0)),
            pl.BlockSpec((1, blk, D), lambda i: (1, i, 0)),
            pl.BlockSpec((NW, blk, 1), lambda i: (0, i, 0)),
        ],
        out_specs=pl.BlockSpec((blk, d_out), lambda i: (i, 0)),
        out_shape=jax.ShapeDtypeStruct((n, d_out), jnp.float32),
    )(acc, acc, den3)
    return h


# speedup vs baseline: 10.4568x; 1.0679x over previous
import jax
import jax.numpy as jnp
from jax import lax
from jax.experimental import pallas as pl
from jax.experimental.pallas import tpu as pltpu
from jax.experimental.pallas import tpu_sc as plsc

D = 128
NC = 2
NS = 16
LANES = 16
NW = NC * NS
NPAD = 10112
STRIPE = NPAD // NS
DENP = 10112
EPW = 10112
CHUNK = 64
NCHUNKS = EPW // CHUNK


def _mm_body(x_ref, wt_ref, a1_ref, a2_ref, z_ref, al1_ref, al2_ref):
    z = jnp.dot(x_ref[...], wt_ref[...], preferred_element_type=jnp.float32)
    z_ref[...] = z
    al1_ref[...] = jnp.dot(z, a1_ref[...], preferred_element_type=jnp.float32)
    al2_ref[...] = jnp.dot(z, a2_ref[...], preferred_element_type=jnp.float32)


def _sc_body(z_hbm, al1_hbm, al2_hbm, src_hbm, dst_hbm, acc_hbm, den_hbm,
             al1_v, al2_v, sidx_v, didx_v, rows_v, w_v, den_v,
             gsem0, gsem1, isem0, isem1, ssem0, ssem1, acc_s):
    cid = lax.axis_index("c")
    sid = lax.axis_index("s")
    wid = cid * NS + sid
    nchunks = src_hbm.shape[1]
    chunk = src_hbm.shape[2]
    gsems = (gsem0, gsem1)
    isems = (isem0, isem1)
    ssems = (ssem0, ssem1)

    zeros16 = jnp.zeros((LANES,), jnp.float32)

    def zero_rows(c, _):
        for k in range(D // LANES):
            rows_v[0, c, pl.ds(k * LANES, LANES)] = zeros16
        return 0
    lax.fori_loop(0, chunk, zero_rows, 0)

    def zero_den(c, _):
        den_v[pl.ds(c * LANES, LANES)] = zeros16
        return 0
    lax.fori_loop(0, DENP // LANES, zero_den, 0)

    base = sid * STRIPE
    off = 0
    while off < STRIPE:
        cnt = min(chunk, STRIPE - off)
        pltpu.sync_copy(rows_v.at[0, pl.ds(0, cnt)],
                        acc_s.at[pl.ds(base + off, cnt)])
        off += cnt

    pltpu.sync_copy(al1_hbm, al1_v)
    pltpu.sync_copy(al2_hbm, al2_v)
    pltpu.sync_copy(src_hbm.at[wid, 0], sidx_v.at[0])
    pltpu.sync_copy(dst_hbm.at[wid, 0], didx_v.at[0])
    pltpu.async_copy(src_hbm.at[wid, 1], sidx_v.at[1], isem1)
    pltpu.async_copy(dst_hbm.at[wid, 1], didx_v.at[1], isem1)
    pltpu.async_copy(z_hbm.at[sidx_v.at[0]], rows_v.at[0], gsem0)

    plsc.subcore_barrier()

    def pipe_step(j, b, last):
        nb = 1 - b
        if not last:
            @pl.when(jnp.asarray(j) > 0)
            def _():
                pltpu.make_async_copy(rows_v.at[nb],
                                      acc_s.at[didx_v.at[nb]],
                                      ssems[nb]).wait()
            pltpu.make_async_copy(src_hbm.at[wid, j + 1],
                                  sidx_v.at[nb], isems[nb]).wait()
            pltpu.make_async_copy(dst_hbm.at[wid, j + 1],
                                  didx_v.at[nb], isems[nb]).wait()
            pltpu.async_copy(z_hbm.at[sidx_v.at[nb]], rows_v.at[nb],
                             gsems[nb])

        pltpu.make_async_copy(z_hbm.at[sidx_v.at[b]], rows_v.at[b],
                              gsems[b]).wait()

        for i in range(chunk // LANES):
            sl = pl.ds(i * LANES, LANES)
            sidx = sidx_v[b, sl]
            didx = didx_v[b, sl]
            a1 = plsc.load_gather(al1_v, [sidx])
            a2 = plsc.load_gather(al2_v, [didx])
            e = a1 + a2
            e = jnp.where(e >= 0.0, e, 0.01 * e)
            w = jnp.exp(e)
            w_v[sl] = w
            plsc.addupdate_scatter(den_v, [didx], w)

        def scale_row(c, _):
            wc = plsc.load_gather(w_v, [jnp.full((LANES,), c, jnp.int32)])
            for k in range(D // LANES):
                sl = pl.ds(k * LANES, LANES)
                rows_v[b, c, sl] = rows_v[b, c, sl] * wc
            return 0
        lax.fori_loop(0, chunk, scale_row, 0)

        pltpu.async_copy(rows_v.at[b], acc_s.at[didx_v.at[b]], ssems[b],
                         add=True)

        if not last:
            @pl.when(jnp.asarray(j) + 2 < nchunks)
            def _():
                pltpu.async_copy(src_hbm.at[wid, j + 2], sidx_v.at[b],
                                 isems[b])
                pltpu.async_copy(dst_hbm.at[wid, j + 2], didx_v.at[b],
                                 isems[b])

    def pair_body(j2, _):
        j = 2 * j2
        pipe_step(j, 0, False)
        pipe_step(j + 1, 1, False)
        return 0

    pairs = (nchunks - 1) // 2
    lax.fori_loop(0, pairs, pair_body, 0)
    for j in range(2 * pairs, nchunks):
        pipe_step(j, j % 2, j == nchunks - 1)
    for b in (0, 1):
        pltpu.make_async_copy(rows_v.at[b], acc_s.at[didx_v.at[b]],
                              ssems[b]).wait()

    pltpu.sync_copy(den_v, den_hbm.at[wid])

    plsc.subcore_barrier()

    pltpu.sync_copy(acc_s.at[pl.ds(base, STRIPE)],
                    acc_hbm.at[cid, pl.ds(base, STRIPE)])


def _combine_body(a0_ref, a1_ref, dp_ref, o_ref):
    s = a0_ref[0] + a1_ref[0]
    den = jnp.sum(dp_ref[...], axis=0)
    o_ref[...] = s / den


def kernel(x, edge_index, W, A):
    n, d_in = x.shape
    d_out = W.shape[0]
    e_total = edge_index.shape[1]
    nchunks = NCHUNKS
    chunk = CHUNK

    wt = W.T
    a1 = A[0, :d_out]
    a2 = A[0, d_out:]

    z, al1, al2 = pl.pallas_call(
        _mm_body,
        out_shape=[
            jax.ShapeDtypeStruct((n, d_out), jnp.float32),
            jax.ShapeDtypeStruct((n,), jnp.float32),
            jax.ShapeDtypeStruct((n,), jnp.float32),
        ],
    )(x, wt, a1, a2)

    npad_e = NW * EPW - e_total
    al1p = jnp.concatenate([al1, jnp.zeros((DENP - n,), jnp.float32)])
    al2p = jnp.concatenate([al2, jnp.zeros((DENP - n,), jnp.float32)])
    src = jnp.concatenate(
        [edge_index[0].astype(jnp.int32),
         jnp.zeros((npad_e,), jnp.int32)]).reshape(NW, nchunks, chunk)
    dst = jnp.concatenate(
        [edge_index[1].astype(jnp.int32),
         jnp.full((npad_e,), DENP - 1, jnp.int32)]).reshape(NW, nchunks, chunk)

    sc = pl.kernel(
        _sc_body,
        out_type=[
            jax.ShapeDtypeStruct((NC, NPAD, D), jnp.float32),
            jax.ShapeDtypeStruct((NW, DENP), jnp.float32),
        ],
        mesh=plsc.VectorSubcoreMesh(
            core_axis_name="c", subcore_axis_name="s",
            num_cores=NC, num_subcores=NS),
        compiler_params=pltpu.CompilerParams(needs_layout_passes=False),
        scratch_types=[
            pltpu.VMEM((DENP,), jnp.float32),
            pltpu.VMEM((DENP,), jnp.float32),
            pltpu.VMEM((2, chunk), jnp.int32),
            pltpu.VMEM((2, chunk), jnp.int32),
            pltpu.VMEM((2, chunk, D), jnp.float32),
            pltpu.VMEM((chunk,), jnp.float32),
            pltpu.VMEM((DENP,), jnp.float32),
            pltpu.SemaphoreType.DMA,
            pltpu.SemaphoreType.DMA,
            pltpu.SemaphoreType.DMA,
            pltpu.SemaphoreType.DMA,
            pltpu.SemaphoreType.DMA,
            pltpu.SemaphoreType.DMA,
            pltpu.VMEM_SHARED((NPAD, D), jnp.float32),
        ],
    )
    acc, den_part = sc(z, al1p, al2p, src, dst)
    den3 = den_part.reshape(NW, DENP, 1)

    blk = 1000
    h = pl.pallas_call(
        _combine_body,
        grid=(n // blk,),
        in_specs=[
            pl.BlockSpec((1, blk, D), lambda i: (0, i, 0)),
            pl.BlockSpec((1, blk, D), lambda i: (1, i, 0)),
            pl.BlockSpec((NW, blk, 1), lambda i: (0, i, 0)),
        ],
        out_specs=pl.BlockSpec((blk, d_out), lambda i: (i, 0)),
        out_shape=jax.ShapeDtypeStruct((n, d_out), jnp.float32),
    )(acc, acc, den3)
    return h


# speedup vs baseline: 10.6154x; 1.0152x over previous
import jax
import jax.numpy as jnp
from jax import lax
from jax.experimental import pallas as pl
from jax.experimental.pallas import tpu as pltpu
from jax.experimental.pallas import tpu_sc as plsc

D = 128
NC = 2
NS = 16
LANES = 16
NW = NC * NS
NPAD = 10112
STRIPE = NPAD // NS
DENP = 10112
EPW = 10112
CHUNK = 64
NCHUNKS = EPW // CHUNK


def _mm_body(x_ref, wt_ref, a1_ref, a2_ref, z_ref, al1_ref, al2_ref):
    z = jnp.dot(x_ref[...], wt_ref[...], preferred_element_type=jnp.float32)
    z_ref[...] = z
    al1_ref[...] = jnp.dot(z, a1_ref[...], preferred_element_type=jnp.float32)
    al2_ref[...] = jnp.dot(z, a2_ref[...], preferred_element_type=jnp.float32)


def _sc_body(z_hbm, al1_hbm, al2_hbm, src_hbm, dst_hbm, acc_hbm, den_hbm,
             al1_v, al2_v, sidx_v, didx_v, rows_v, w_v, den_v,
             gsem0, gsem1, isem0, isem1, ssem0, ssem1, acc_s):
    cid = lax.axis_index("c")
    sid = lax.axis_index("s")
    wid = cid * NS + sid
    nchunks = src_hbm.shape[1]
    chunk = src_hbm.shape[2]
    gsems = (gsem0, gsem1)
    isems = (isem0, isem1)
    ssems = (ssem0, ssem1)

    zeros16 = jnp.zeros((LANES,), jnp.float32)

    def zero_rows(c, _):
        for k in range(D // LANES):
            rows_v[0, c, pl.ds(k * LANES, LANES)] = zeros16
        return 0
    lax.fori_loop(0, chunk, zero_rows, 0)

    def zero_den(c, _):
        den_v[pl.ds(c * LANES, LANES)] = zeros16
        return 0
    lax.fori_loop(0, DENP // LANES, zero_den, 0)

    base = sid * STRIPE
    off = 0
    while off < STRIPE:
        cnt = min(chunk, STRIPE - off)
        pltpu.sync_copy(rows_v.at[0, pl.ds(0, cnt)],
                        acc_s.at[pl.ds(base + off, cnt)])
        off += cnt

    pltpu.sync_copy(al1_hbm, al1_v)
    pltpu.sync_copy(al2_hbm, al2_v)
    pltpu.sync_copy(src_hbm.at[wid, 0], sidx_v.at[0])
    pltpu.sync_copy(dst_hbm.at[wid, 0], didx_v.at[0])
    pltpu.async_copy(src_hbm.at[wid, 1], sidx_v.at[1], isem1)
    pltpu.async_copy(dst_hbm.at[wid, 1], didx_v.at[1], isem1)
    pltpu.async_copy(z_hbm.at[sidx_v.at[0]], rows_v.at[0], gsem0)

    plsc.subcore_barrier()

    def pipe_step(j, b, last):
        nb = 1 - b
        if not last:
            @pl.when(jnp.asarray(j) > 0)
            def _():
                pltpu.make_async_copy(rows_v.at[nb],
                                      acc_s.at[didx_v.at[nb]],
                                      ssems[nb]).wait()
            pltpu.make_async_copy(src_hbm.at[wid, j + 1],
                                  sidx_v.at[nb], isems[nb]).wait()
            pltpu.make_async_copy(dst_hbm.at[wid, j + 1],
                                  didx_v.at[nb], isems[nb]).wait()
            pltpu.async_copy(z_hbm.at[sidx_v.at[nb]], rows_v.at[nb],
                             gsems[nb])

        pltpu.make_async_copy(z_hbm.at[sidx_v.at[b]], rows_v.at[b],
                              gsems[b]).wait()

        for i in range(chunk // LANES):
            sl = pl.ds(i * LANES, LANES)
            sidx = sidx_v[b, sl]
            didx = didx_v[b, sl]
            a1 = plsc.load_gather(al1_v, [sidx])
            a2 = plsc.load_gather(al2_v, [didx])
            e = a1 + a2
            e = jnp.where(e >= 0.0, e, 0.01 * e)
            w = jnp.exp(e)
            w_v[sl] = w
            plsc.addupdate_scatter(den_v, [didx], w)

        def scale_rows(c4, _):
            c0 = c4 * 4
            wcs = [plsc.load_gather(
                w_v, [jnp.full((LANES,), c0 + r, jnp.int32)])
                for r in range(4)]
            for k in range(D // LANES):
                sl = pl.ds(k * LANES, LANES)
                for r in range(4):
                    rows_v[b, c0 + r, sl] = rows_v[b, c0 + r, sl] * wcs[r]
            return 0
        lax.fori_loop(0, chunk // 4, scale_rows, 0)

        pltpu.async_copy(rows_v.at[b], acc_s.at[didx_v.at[b]], ssems[b],
                         add=True)

        if not last:
            @pl.when(jnp.asarray(j) + 2 < nchunks)
            def _():
                pltpu.async_copy(src_hbm.at[wid, j + 2], sidx_v.at[b],
                                 isems[b])
                pltpu.async_copy(dst_hbm.at[wid, j + 2], didx_v.at[b],
                                 isems[b])

    def pair_body(j2, _):
        j = 2 * j2
        pipe_step(j, 0, False)
        pipe_step(j + 1, 1, False)
        return 0

    pairs = (nchunks - 1) // 2
    lax.fori_loop(0, pairs, pair_body, 0)
    for j in range(2 * pairs, nchunks):
        pipe_step(j, j % 2, j == nchunks - 1)
    for b in (0, 1):
        pltpu.make_async_copy(rows_v.at[b], acc_s.at[didx_v.at[b]],
                              ssems[b]).wait()

    pltpu.sync_copy(den_v, den_hbm.at[wid])

    plsc.subcore_barrier()

    pltpu.sync_copy(acc_s.at[pl.ds(base, STRIPE)],
                    acc_hbm.at[cid, pl.ds(base, STRIPE)])


def _combine_body(a0_ref, a1_ref, dp_ref, o_ref):
    s = a0_ref[0] + a1_ref[0]
    den = jnp.sum(dp_ref[...], axis=0)
    o_ref[...] = s / den


def kernel(x, edge_index, W, A):
    n, d_in = x.shape
    d_out = W.shape[0]
    e_total = edge_index.shape[1]
    nchunks = NCHUNKS
    chunk = CHUNK

    wt = W.T
    a1 = A[0, :d_out]
    a2 = A[0, d_out:]

    z, al1, al2 = pl.pallas_call(
        _mm_body,
        out_shape=[
            jax.ShapeDtypeStruct((n, d_out), jnp.float32),
            jax.ShapeDtypeStruct((n,), jnp.float32),
            jax.ShapeDtypeStruct((n,), jnp.float32),
        ],
    )(x, wt, a1, a2)

    npad_e = NW * EPW - e_total
    al1p = jnp.concatenate([al1, jnp.zeros((DENP - n,), jnp.float32)])
    al2p = jnp.concatenate([al2, jnp.zeros((DENP - n,), jnp.float32)])
    src = jnp.concatenate(
        [edge_index[0].astype(jnp.int32),
         jnp.zeros((npad_e,), jnp.int32)]).reshape(NW, nchunks, chunk)
    dst = jnp.concatenate(
        [edge_index[1].astype(jnp.int32),
         jnp.full((npad_e,), DENP - 1, jnp.int32)]).reshape(NW, nchunks, chunk)

    sc = pl.kernel(
        _sc_body,
        out_type=[
            jax.ShapeDtypeStruct((NC, NPAD, D), jnp.float32),
            jax.ShapeDtypeStruct((NW, DENP), jnp.float32),
        ],
        mesh=plsc.VectorSubcoreMesh(
            core_axis_name="c", subcore_axis_name="s",
            num_cores=NC, num_subcores=NS),
        compiler_params=pltpu.CompilerParams(needs_layout_passes=False),
        scratch_types=[
            pltpu.VMEM((DENP,), jnp.float32),
            pltpu.VMEM((DENP,), jnp.float32),
            pltpu.VMEM((2, chunk), jnp.int32),
            pltpu.VMEM((2, chunk), jnp.int32),
            pltpu.VMEM((2, chunk, D), jnp.float32),
            pltpu.VMEM((chunk,), jnp.float32),
            pltpu.VMEM((DENP,), jnp.float32),
            pltpu.SemaphoreType.DMA,
            pltpu.SemaphoreType.DMA,
            pltpu.SemaphoreType.DMA,
            pltpu.SemaphoreType.DMA,
            pltpu.SemaphoreType.DMA,
            pltpu.SemaphoreType.DMA,
            pltpu.VMEM_SHARED((NPAD, D), jnp.float32),
        ],
    )
    acc, den_part = sc(z, al1p, al2p, src, dst)
    den3 = den_part.reshape(NW, DENP, 1)

    blk = 1000
    h = pl.pallas_call(
        _combine_body,
        grid=(n // blk,),
        in_specs=[
            pl.BlockSpec((1, blk, D), lambda i: (0, i, 0)),
            pl.BlockSpec((1, blk, D), lambda i: (1, i, 0)),
            pl.BlockSpec((NW, blk, 1), lambda i: (0, i, 0)),
        ],
        out_specs=pl.BlockSpec((blk, d_out), lambda i: (i, 0)),
        out_shape=jax.ShapeDtypeStruct((n, d_out), jnp.float32),
    )(acc, acc, den3)
    return h


# speedup vs baseline: 10.8319x; 1.0204x over previous
import jax
import jax.numpy as jnp
from jax import lax
from jax.experimental import pallas as pl
from jax.experimental.pallas import tpu as pltpu
from jax.experimental.pallas import tpu_sc as plsc

D = 128
NC = 2
NS = 16
LANES = 16
NW = NC * NS
NPAD = 10112
STRIPE = NPAD // NS
DENP = 10112
EPW = 10112
CHUNK = 64
NCHUNKS = EPW // CHUNK


def _mm_body(x_ref, wt_ref, a1_ref, a2_ref, z_ref, al1_ref, al2_ref):
    z = jnp.dot(x_ref[...], wt_ref[...], preferred_element_type=jnp.float32)
    z_ref[...] = z
    al1_ref[...] = jnp.dot(z, a1_ref[...], preferred_element_type=jnp.float32)
    al2_ref[...] = jnp.dot(z, a2_ref[...], preferred_element_type=jnp.float32)


def _sc_body(z_hbm, al1_hbm, al2_hbm, edges_hbm, acc_hbm, den_hbm,
             al1_v, al2_v, idx_v, rows_v, w_v, den_v,
             gsem0, gsem1, isem0, isem1, isem2, ssem0, ssem1, acc_s):
    cid = lax.axis_index("c")
    sid = lax.axis_index("s")
    wid = cid * NS + sid
    npairs = edges_hbm.shape[1]
    chunk = edges_hbm.shape[4]
    nchunks = 2 * npairs
    gsems = (gsem0, gsem1)
    isems = (isem0, isem1, isem2)
    ssems = (ssem0, ssem1)

    zeros16 = jnp.zeros((LANES,), jnp.float32)

    def zero_rows(c, _):
        for k in range(D // LANES):
            rows_v[0, c, pl.ds(k * LANES, LANES)] = zeros16
        return 0
    lax.fori_loop(0, chunk, zero_rows, 0)

    def zero_den(c, _):
        den_v[pl.ds(c * LANES, LANES)] = zeros16
        return 0
    lax.fori_loop(0, DENP // LANES, zero_den, 0)

    base = sid * STRIPE
    off = 0
    while off < STRIPE:
        cnt = min(chunk, STRIPE - off)
        pltpu.sync_copy(rows_v.at[0, pl.ds(0, cnt)],
                        acc_s.at[pl.ds(base + off, cnt)])
        off += cnt

    pltpu.sync_copy(al1_hbm, al1_v)
    pltpu.sync_copy(al2_hbm, al2_v)
    pltpu.sync_copy(edges_hbm.at[wid, 0], idx_v.at[0])
    pltpu.async_copy(edges_hbm.at[wid, 1], idx_v.at[1], isem1)
    pltpu.async_copy(z_hbm.at[idx_v.at[0, 0, 0]], rows_v.at[0], gsem0)

    plsc.subcore_barrier()

    def pipe_step(j, b, ps, nps, first, last):
        nb = 1 - b
        if not last:
            def drain_nb():
                pltpu.make_async_copy(rows_v.at[nb],
                                      acc_s.at[idx_v.at[0, 0, 1]],
                                      ssems[nb]).wait()
            if first:
                pass
            else:
                drain_nb()
            pltpu.async_copy(z_hbm.at[idx_v.at[nps, (b + 1) % 2, 0]],
                             rows_v.at[nb], gsems[nb])

        pltpu.make_async_copy(z_hbm.at[idx_v.at[ps, b, 0]], rows_v.at[b],
                              gsems[b]).wait()

        for i in range(chunk // LANES):
            sl = pl.ds(i * LANES, LANES)
            sidx = idx_v[ps, b, 0, sl]
            didx = idx_v[ps, b, 1, sl]
            a1 = plsc.load_gather(al1_v, [sidx])
            a2 = plsc.load_gather(al2_v, [didx])
            e = a1 + a2
            e = jnp.where(e >= 0.0, e, 0.01 * e)
            w = jnp.exp(e)
            w_v[sl] = w
            plsc.addupdate_scatter(den_v, [didx], w)

        def scale_rows(c4, _):
            c0 = c4 * 4
            wcs = [plsc.load_gather(
                w_v, [jnp.full((LANES,), c0 + r, jnp.int32)])
                for r in range(4)]
            for k in range(D // LANES):
                sl = pl.ds(k * LANES, LANES)
                for r in range(4):
                    rows_v[b, c0 + r, sl] = rows_v[b, c0 + r, sl] * wcs[r]
            return 0
        lax.fori_loop(0, chunk // 4, scale_rows, 0)

        pltpu.async_copy(rows_v.at[b], acc_s.at[idx_v.at[ps, b, 1]],
                         ssems[b], add=True)

    def pair_step(p, ps, first, last):
        ps1 = (ps + 1) % 3
        ps2 = (ps + 2) % 3
        pipe_step(2 * p, 0, ps, ps, first, False)
        if not last:
            pltpu.make_async_copy(edges_hbm.at[wid, p + 1],
                                  idx_v.at[ps1], isems[ps1]).wait()
            pipe_step(2 * p + 1, 1, ps, ps1, False, False)
            @pl.when(jnp.asarray(p) + 2 < npairs)
            def _():
                pltpu.async_copy(edges_hbm.at[wid, p + 2], idx_v.at[ps2],
                                 isems[ps2])
        else:
            pipe_step(2 * p + 1, 1, ps, ps, False, True)

    def triple_body(q, _):
        p = 3 * q + 1
        pair_step(p, 1, False, False)
        pair_step(p + 1, 2, False, False)
        pair_step(p + 2, 0, False, False)
        return 0

    pair_step(0, 0, True, False)
    triples = (npairs - 2) // 3
    lax.fori_loop(0, triples, triple_body, 0)
    for p in range(1 + 3 * triples, npairs):
        pair_step(p, p % 3, False, p == npairs - 1)
    for b in (0, 1):
        pltpu.make_async_copy(rows_v.at[b], acc_s.at[idx_v.at[0, 0, 1]],
                              ssems[b]).wait()

    pltpu.sync_copy(den_v, den_hbm.at[wid])

    plsc.subcore_barrier()

    pltpu.sync_copy(acc_s.at[pl.ds(base, STRIPE)],
                    acc_hbm.at[cid, pl.ds(base, STRIPE)])


def _combine_body(a0_ref, a1_ref, dp_ref, o_ref):
    s = a0_ref[0] + a1_ref[0]
    den = jnp.sum(dp_ref[...], axis=0)
    o_ref[...] = s / den


def kernel(x, edge_index, W, A):
    n, d_in = x.shape
    d_out = W.shape[0]
    e_total = edge_index.shape[1]
    nchunks = NCHUNKS
    chunk = CHUNK

    wt = W.T
    a1 = A[0, :d_out]
    a2 = A[0, d_out:]

    z, al1, al2 = pl.pallas_call(
        _mm_body,
        out_shape=[
            jax.ShapeDtypeStruct((n, d_out), jnp.float32),
            jax.ShapeDtypeStruct((n,), jnp.float32),
            jax.ShapeDtypeStruct((n,), jnp.float32),
        ],
    )(x, wt, a1, a2)

    npad_e = NW * EPW - e_total
    al1p = jnp.concatenate([al1, jnp.zeros((DENP - n,), jnp.float32)])
    al2p = jnp.concatenate([al2, jnp.zeros((DENP - n,), jnp.float32)])
    src = jnp.concatenate(
        [edge_index[0].astype(jnp.int32),
         jnp.zeros((npad_e,), jnp.int32)]).reshape(NW, nchunks, 1, chunk)
    dst = jnp.concatenate(
        [edge_index[1].astype(jnp.int32),
         jnp.full((npad_e,), DENP - 1, jnp.int32)]).reshape(
             NW, nchunks, 1, chunk)
    edges = jnp.concatenate([src, dst], axis=2).reshape(
        NW, nchunks // 2, 2, 2, chunk)

    sc = pl.kernel(
        _sc_body,
        out_type=[
            jax.ShapeDtypeStruct((NC, NPAD, D), jnp.float32),
            jax.ShapeDtypeStruct((NW, DENP), jnp.float32),
        ],
        mesh=plsc.VectorSubcoreMesh(
            core_axis_name="c", subcore_axis_name="s",
            num_cores=NC, num_subcores=NS),
        compiler_params=pltpu.CompilerParams(needs_layout_passes=False),
        scratch_types=[
            pltpu.VMEM((DENP,), jnp.float32),
            pltpu.VMEM((DENP,), jnp.float32),
            pltpu.VMEM((3, 2, 2, chunk), jnp.int32),
            pltpu.VMEM((2, chunk, D), jnp.float32),
            pltpu.VMEM((chunk,), jnp.float32),
            pltpu.VMEM((DENP,), jnp.float32),
            pltpu.SemaphoreType.DMA,
            pltpu.SemaphoreType.DMA,
            pltpu.SemaphoreType.DMA,
            pltpu.SemaphoreType.DMA,
            pltpu.SemaphoreType.DMA,
            pltpu.SemaphoreType.DMA,
            pltpu.SemaphoreType.DMA,
            pltpu.VMEM_SHARED((NPAD, D), jnp.float32),
        ],
    )
    acc, den_part = sc(z, al1p, al2p, edges)
    den3 = den_part.reshape(NW, DENP, 1)

    blk = 1000
    h = pl.pallas_call(
        _combine_body,
        grid=(n // blk,),
        in_specs=[
            pl.BlockSpec((1, blk, D), lambda i: (0, i, 0)),
            pl.BlockSpec((1, blk, D), lambda i: (1, i, 0)),
            pl.BlockSpec((NW, blk, 1), lambda i: (0, i, 0)),
        ],
        out_specs=pl.BlockSpec((blk, d_out), lambda i: (i, 0)),
        out_shape=jax.ShapeDtypeStruct((n, d_out), jnp.float32),
    )(acc, acc, den3)
    return h
